# baseline (speedup 1.0000x reference)
import functools

import jax
import jax.numpy as jnp
from jax.experimental import pallas as pl


def _body(x_ref, cnt_ref, w1bd_ref, b1bd_ref, w2_ref, b2_ref, w3_ref,
          b3_ref, out_ref, *, tile, P, C):
    cnt = cnt_ref[0]
    x = x_ref[...]
    lane = jax.lax.broadcasted_iota(jnp.int32, x.shape, 1)
    x = jnp.where((cnt == 0) & (lane < C), 0.0, x)

    a = jnp.dot(x.astype(jnp.bfloat16), w1bd_ref[...],
                preferred_element_type=jnp.float32) + b1bd_ref[...]
    a = jnp.maximum(a, 0.0)
    h = a.reshape(tile * 4, P * 8)
    h = jnp.maximum(
        jnp.dot(h.astype(jnp.bfloat16), w2_ref[...],
                preferred_element_type=jnp.float32) + b2_ref[...], 0.0)
    h = h.reshape(tile * P, h.shape[-1] // 8)
    h = jnp.dot(h.astype(jnp.bfloat16), w3_ref[...],
                preferred_element_type=jnp.float32)
    h = h.reshape(tile, P, h.shape[-1])

    pen2d = jnp.where(
        jax.lax.broadcasted_iota(jnp.int32, (tile, P), 1)
        < jnp.maximum(cnt, 1),
        0.0, jnp.float32(-1e30))
    h = h + jax.lax.broadcast_in_dim(pen2d, (tile, P, h.shape[-1]), (0, 1))
    out_ref[...] = jnp.max(h, axis=1) + b3_ref[...]


def kernel(Frustum_Voxel, Frustum_Voxel_num, W1, b1, W2, b2, W3, b3):
    B, H, Wd, P, C = Frustum_Voxel.shape
    N = B * H * Wd
    D = W3.shape[1]
    tile = 128
    nt = N // tile

    x = Frustum_Voxel.reshape(N, P * C)
    cnt3 = Frustum_Voxel_num.reshape(nt, tile, 1)
    w1bd = jnp.kron(jnp.eye(P, dtype=W1.dtype), W1)
    b1bd = jnp.tile(b1, P).reshape(1, -1)
    w2bd = jnp.kron(jnp.eye(8, dtype=W2.dtype), W2)
    b2bd = jnp.tile(b2, 8).reshape(1, -1)

    body = functools.partial(_body, tile=tile, P=P, C=C)
    out = pl.pallas_call(
        body,
        grid=(nt,),
        in_specs=[
            pl.BlockSpec((tile, P * C), lambda i: (i, 0)),
            pl.BlockSpec((1, tile, 1), lambda i: (i, 0, 0)),
            pl.BlockSpec((P * C, P * 32), lambda i: (0, 0)),
            pl.BlockSpec((1, P * 32), lambda i: (0, 0)),
            pl.BlockSpec((P * 8, 8 * W2.shape[1]), lambda i: (0, 0)),
            pl.BlockSpec((1, 8 * b2.shape[0]), lambda i: (0, 0)),
            pl.BlockSpec(W3.shape, lambda i: (0, 0)),
            pl.BlockSpec((1, b3.shape[0]), lambda i: (0, 0)),
        ],
        out_specs=pl.BlockSpec((tile, D), lambda i: (i, 0)),
        out_shape=jax.ShapeDtypeStruct((N, D), jnp.float32),
    )(x, cnt3, w1bd.astype(jnp.bfloat16), b1bd,
      w2bd.astype(jnp.bfloat16), b2bd,
      W3.astype(jnp.bfloat16), b3.reshape(1, -1))
    return out.reshape(B, H, Wd, D)


# speedup vs baseline: 1.0511x; 1.0511x over previous
import functools

import jax
import jax.numpy as jnp
from jax.experimental import pallas as pl


def _mlp_tail(h, w2, b2, w3, b3):
    h = jnp.maximum(
        jnp.dot(h.astype(jnp.bfloat16), w2,
                preferred_element_type=jnp.float32) + b2, 0.0)
    return jnp.dot(h.astype(jnp.bfloat16), w3,
                   preferred_element_type=jnp.float32)


def _body(pts_ref, cnt_ref, w1_ref, b1_ref, w2_ref, b2_ref, w3_ref,
          b3_ref, out_ref, *, tile, P):
    pts = pts_ref[...].astype(jnp.bfloat16)
    b1 = b1_ref[...]
    h = jnp.maximum(
        jnp.dot(pts, w1_ref[...], preferred_element_type=jnp.float32)
        + b1, 0.0)
    h = _mlp_tail(h, w2_ref[...], b2_ref[...], w3_ref[...], b3_ref[...])
    h = h.reshape(tile, P, h.shape[-1])

    cnt = cnt_ref[0]
    pen2d = jnp.where(
        jax.lax.broadcasted_iota(jnp.int32, (tile, P), 1) < cnt,
        0.0, jnp.float32(-1e30))
    h = h + jax.lax.broadcast_in_dim(pen2d, (tile, P, h.shape[-1]), (0, 1))
    feat = jnp.max(h, axis=1)

    z = jnp.maximum(b1, 0.0)
    z = _mlp_tail(z, w2_ref[...], b2_ref[...], w3_ref[...], b3_ref[...])
    out_ref[...] = jnp.where(cnt > 0, feat, z) + b3_ref[...]


def kernel(Frustum_Voxel, Frustum_Voxel_num, W1, b1, W2, b2, W3, b3):
    B, H, Wd, P, C = Frustum_Voxel.shape
    N = B * H * Wd
    D = W3.shape[1]
    tile = 128
    nt = N // tile

    pts = Frustum_Voxel.reshape(N * P, C)
    cnt3 = Frustum_Voxel_num.reshape(nt, tile, 1)

    body = functools.partial(_body, tile=tile, P=P)
    out = pl.pallas_call(
        body,
        grid=(nt,),
        in_specs=[
            pl.BlockSpec((tile * P, C), lambda i: (i, 0)),
            pl.BlockSpec((1, tile, 1), lambda i: (i, 0, 0)),
            pl.BlockSpec(W1.shape, lambda i: (0, 0)),
            pl.BlockSpec((1, b1.shape[0]), lambda i: (0, 0)),
            pl.BlockSpec(W2.shape, lambda i: (0, 0)),
            pl.BlockSpec((1, b2.shape[0]), lambda i: (0, 0)),
            pl.BlockSpec(W3.shape, lambda i: (0, 0)),
            pl.BlockSpec((1, b3.shape[0]), lambda i: (0, 0)),
        ],
        out_specs=pl.BlockSpec((tile, D), lambda i: (i, 0)),
        out_shape=jax.ShapeDtypeStruct((N, D), jnp.float32),
    )(pts, cnt3, W1.astype(jnp.bfloat16), b1.reshape(1, -1),
      W2.astype(jnp.bfloat16), b2.reshape(1, -1),
      W3.astype(jnp.bfloat16), b3.reshape(1, -1))
    return out.reshape(B, H, Wd, D)


# speedup vs baseline: 1.0960x; 1.0427x over previous
import functools

import jax
import jax.numpy as jnp
from jax.experimental import pallas as pl


def _mlp_tail(h, w2, b2, w3, b3):
    h = jnp.maximum(
        jnp.dot(h.astype(jnp.bfloat16), w2,
                preferred_element_type=jnp.float32) + b2, 0.0)
    return jnp.dot(h.astype(jnp.bfloat16), w3,
                   preferred_element_type=jnp.float32)


def _body(pts_ref, cnt_ref, w1_ref, b1_ref, w2_ref, b2_ref, w3_ref,
          b3_ref, out_ref, *, tile, P):
    pts = pts_ref[...].astype(jnp.bfloat16)
    b1 = b1_ref[...]
    h = jnp.maximum(
        jnp.dot(pts, w1_ref[...], preferred_element_type=jnp.float32)
        + b1, 0.0)
    h = _mlp_tail(h, w2_ref[...], b2_ref[...], w3_ref[...], b3_ref[...])
    h = h.reshape(tile, P, h.shape[-1])

    cnt = cnt_ref[0]
    pen2d = jnp.where(
        jax.lax.broadcasted_iota(jnp.int32, (tile, P), 1) < cnt,
        0.0, jnp.float32(-1e30))
    h = h + jax.lax.broadcast_in_dim(pen2d, (tile, P, h.shape[-1]), (0, 1))
    feat = jnp.max(h, axis=1)

    z = jnp.maximum(b1, 0.0)
    z = _mlp_tail(z, w2_ref[...], b2_ref[...], w3_ref[...], b3_ref[...])
    out_ref[...] = jnp.where(cnt > 0, feat, z) + b3_ref[...]


def kernel(Frustum_Voxel, Frustum_Voxel_num, W1, b1, W2, b2, W3, b3):
    B, H, Wd, P, C = Frustum_Voxel.shape
    N = B * H * Wd
    D = W3.shape[1]
    tile = 256
    nt = N // tile

    pts = Frustum_Voxel.reshape(N * P, C)
    cnt3 = Frustum_Voxel_num.reshape(nt, tile, 1)

    body = functools.partial(_body, tile=tile, P=P)
    out = pl.pallas_call(
        body,
        grid=(nt,),
        in_specs=[
            pl.BlockSpec((tile * P, C), lambda i: (i, 0)),
            pl.BlockSpec((1, tile, 1), lambda i: (i, 0, 0)),
            pl.BlockSpec(W1.shape, lambda i: (0, 0)),
            pl.BlockSpec((1, b1.shape[0]), lambda i: (0, 0)),
            pl.BlockSpec(W2.shape, lambda i: (0, 0)),
            pl.BlockSpec((1, b2.shape[0]), lambda i: (0, 0)),
            pl.BlockSpec(W3.shape, lambda i: (0, 0)),
            pl.BlockSpec((1, b3.shape[0]), lambda i: (0, 0)),
        ],
        out_specs=pl.BlockSpec((tile, D), lambda i: (i, 0)),
        out_shape=jax.ShapeDtypeStruct((N, D), jnp.float32),
    )(pts, cnt3, W1.astype(jnp.bfloat16), b1.reshape(1, -1),
      W2.astype(jnp.bfloat16), b2.reshape(1, -1),
      W3.astype(jnp.bfloat16), b3.reshape(1, -1))
    return out.reshape(B, H, Wd, D)


# speedup vs baseline: 1.1047x; 1.0079x over previous
import functools

import jax
import jax.numpy as jnp
from jax.experimental import pallas as pl


def _mlp_tail(h, w2, b2, w3, b3):
    h = jnp.maximum(
        jnp.dot(h.astype(jnp.bfloat16), w2,
                preferred_element_type=jnp.float32) + b2, 0.0)
    return jnp.dot(h.astype(jnp.bfloat16), w3,
                   preferred_element_type=jnp.float32)


def _body(pts_ref, cnt_ref, w1_ref, b1_ref, w2_ref, b2_ref, w3_ref,
          b3_ref, out_ref, *, tile, P):
    pts = pts_ref[...].astype(jnp.bfloat16)
    b1 = b1_ref[...]
    h = jnp.maximum(
        jnp.dot(pts, w1_ref[...], preferred_element_type=jnp.float32)
        + b1, 0.0)
    h = _mlp_tail(h, w2_ref[...], b2_ref[...], w3_ref[...], b3_ref[...])
    h = h.reshape(tile, P, h.shape[-1])

    cnt = cnt_ref[0]
    pen2d = jnp.where(
        jax.lax.broadcasted_iota(jnp.int32, (tile, P), 1) < cnt,
        0.0, jnp.float32(-1e30))
    h = h + jax.lax.broadcast_in_dim(pen2d, (tile, P, h.shape[-1]), (0, 1))
    feat = jnp.max(h, axis=1)

    z = jnp.maximum(b1, 0.0)
    z = _mlp_tail(z, w2_ref[...], b2_ref[...], w3_ref[...], b3_ref[...])
    out_ref[...] = jnp.where(cnt > 0, feat, z) + b3_ref[...]


def kernel(Frustum_Voxel, Frustum_Voxel_num, W1, b1, W2, b2, W3, b3):
    B, H, Wd, P, C = Frustum_Voxel.shape
    N = B * H * Wd
    D = W3.shape[1]
    tile = 512
    nt = N // tile

    pts = Frustum_Voxel.reshape(N * P, C)
    cnt3 = Frustum_Voxel_num.reshape(nt, tile, 1)

    body = functools.partial(_body, tile=tile, P=P)
    out = pl.pallas_call(
        body,
        grid=(nt,),
        in_specs=[
            pl.BlockSpec((tile * P, C), lambda i: (i, 0)),
            pl.BlockSpec((1, tile, 1), lambda i: (i, 0, 0)),
            pl.BlockSpec(W1.shape, lambda i: (0, 0)),
            pl.BlockSpec((1, b1.shape[0]), lambda i: (0, 0)),
            pl.BlockSpec(W2.shape, lambda i: (0, 0)),
            pl.BlockSpec((1, b2.shape[0]), lambda i: (0, 0)),
            pl.BlockSpec(W3.shape, lambda i: (0, 0)),
            pl.BlockSpec((1, b3.shape[0]), lambda i: (0, 0)),
        ],
        out_specs=pl.BlockSpec((tile, D), lambda i: (i, 0)),
        out_shape=jax.ShapeDtypeStruct((N, D), jnp.float32),
    )(pts, cnt3, W1.astype(jnp.bfloat16), b1.reshape(1, -1),
      W2.astype(jnp.bfloat16), b2.reshape(1, -1),
      W3.astype(jnp.bfloat16), b3.reshape(1, -1))
    return out.reshape(B, H, Wd, D)
